# initial kernel scaffold (unmeasured)
import functools

import jax
import jax.numpy as jnp
from jax import lax
from jax.experimental import pallas as pl
from jax.experimental.pallas import tpu as pltpu

N_DEV = 4

COMPUTE_IN = "fp8"

BM = 512
BN = 2048


def _ag_body(x_ref, w_ref, xg_ref, wg_ref,
             comm_x, comm_w, send_x, recv_x, send_w, recv_w, copy_sem):
    my = lax.axis_index("i")
    left = lax.rem(my + N_DEV - 1, N_DEV)
    right = lax.rem(my + 1, N_DEV)
    kx = x_ref.shape[1]
    kw = w_ref.shape[0]

    barrier = pltpu.get_barrier_semaphore()
    for nbr in (left, right):
        pl.semaphore_signal(barrier, inc=1, device_id=(nbr,),
                            device_id_type=pl.DeviceIdType.MESH)
    pl.semaphore_wait(barrier, 2)

    own_x = pltpu.make_async_copy(x_ref, xg_ref.at[:, pl.ds(my * kx, kx)],
                                  copy_sem)
    own_x.start()
    own_x.wait()
    own_w = pltpu.make_async_copy(w_ref, wg_ref.at[pl.ds(my * kw, kw), :],
                                  copy_sem)
    own_w.start()
    own_w.wait()
    cx = pltpu.make_async_copy(x_ref, comm_x.at[0], copy_sem)
    cx.start()
    cx.wait()
    cw = pltpu.make_async_copy(w_ref, comm_w.at[0], copy_sem)
    cw.start()
    cw.wait()

    for h in range(N_DEV - 1):
        s, r = h % 2, (h + 1) % 2
        rx = pltpu.make_async_remote_copy(
            src_ref=comm_x.at[s], dst_ref=comm_x.at[r],
            send_sem=send_x.at[s], recv_sem=recv_x.at[r],
            device_id=(right,), device_id_type=pl.DeviceIdType.MESH)
        rw = pltpu.make_async_remote_copy(
            src_ref=comm_w.at[s], dst_ref=comm_w.at[r],
            send_sem=send_w.at[s], recv_sem=recv_w.at[r],
            device_id=(right,), device_id_type=pl.DeviceIdType.MESH)
        rx.start()
        rw.start()
        rx.wait()
        rw.wait()

        origin = lax.rem(my + N_DEV - 1 - h, N_DEV)
        ox = pltpu.make_async_copy(
            comm_x.at[r], xg_ref.at[:, pl.ds(origin * kx, kx)], copy_sem)
        ox.start()
        ox.wait()
        ow = pltpu.make_async_copy(
            comm_w.at[r], wg_ref.at[pl.ds(origin * kw, kw), :], copy_sem)
        ow.start()
        ow.wait()


def _all_gather(xc, wc):
    m, kx = xc.shape
    kw, n = wc.shape
    return pl.pallas_call(
        _ag_body,
        out_shape=[
            jax.ShapeDtypeStruct((m, N_DEV * kx), xc.dtype),
            jax.ShapeDtypeStruct((N_DEV * kw, n), wc.dtype),
        ],
        in_specs=[pl.BlockSpec(memory_space=pltpu.VMEM),
                  pl.BlockSpec(memory_space=pltpu.VMEM)],
        out_specs=[pl.BlockSpec(memory_space=pltpu.ANY),
                   pl.BlockSpec(memory_space=pltpu.ANY)],
        scratch_shapes=[
            pltpu.VMEM((2, m, kx), xc.dtype),
            pltpu.VMEM((2, kw, n), wc.dtype),
            pltpu.SemaphoreType.DMA((2,)),
            pltpu.SemaphoreType.DMA((2,)),
            pltpu.SemaphoreType.DMA((2,)),
            pltpu.SemaphoreType.DMA((2,)),
            pltpu.SemaphoreType.DMA,
        ],
        compiler_params=pltpu.CompilerParams(collective_id=0),
    )(xc, wc)


def _gemm_body(s_ref, x_ref, w_ref, o_ref):
    o_ref[...] = (
        jnp.dot(x_ref[...], w_ref[...], preferred_element_type=jnp.float32)
        * s_ref[0, 0]
    )


def _gemm(s, xg, wg):
    m, k = xg.shape
    _, n = wg.shape
    return pl.pallas_call(
        _gemm_body,
        grid=(m // BM, n // BN),
        in_specs=[
            pl.BlockSpec((1, 1), lambda i, j: (0, 0),
                         memory_space=pltpu.SMEM),
            pl.BlockSpec((BM, k), lambda i, j: (i, 0)),
            pl.BlockSpec((k, BN), lambda i, j: (0, j)),
        ],
        out_specs=pl.BlockSpec((BM, BN), lambda i, j: (i, j)),
        out_shape=jax.ShapeDtypeStruct((m, n), jnp.float32),
        compiler_params=pltpu.CompilerParams(
            dimension_semantics=("parallel", "parallel")),
    )(s, xg, wg)


def kernel(x, w_mat, scale_x, scale_w):
    xc = x.astype(jnp.float8_e4m3fn)
    wc = w_mat.astype(jnp.float8_e5m2)
    xg, wg = _all_gather(xc, wc)
    if COMPUTE_IN == "bf16":
        xg = xg.astype(jnp.bfloat16)
        wg = wg.astype(jnp.bfloat16)
    s = (scale_x * scale_w).reshape(1, 1)
    return _gemm(s, xg, wg)


# baseline (device time: 617679 ns/iter reference)
import functools

import jax
import jax.numpy as jnp
from jax import lax
from jax.experimental import pallas as pl
from jax.experimental.pallas import tpu as pltpu

N_DEV = 4

COMPUTE_IN = "fp8"

BM = 512
BN = 2048


def _ag_body(x_ref, w_ref, xg_ref, wg_ref,
             comm_x, comm_w, send_x, recv_x, send_w, recv_w, copy_sem):
    my = lax.axis_index("i")
    left = lax.rem(my + N_DEV - 1, N_DEV)
    right = lax.rem(my + 1, N_DEV)
    kx = x_ref.shape[1]
    kw = w_ref.shape[0]

    barrier = pltpu.get_barrier_semaphore()
    for nbr in (left, right):
        pl.semaphore_signal(barrier, inc=1, device_id=(nbr,),
                            device_id_type=pl.DeviceIdType.MESH)
    pl.semaphore_wait(barrier, 2)

    own_x = pltpu.make_async_copy(x_ref, xg_ref.at[:, pl.ds(my * kx, kx)],
                                  copy_sem)
    own_x.start()
    own_x.wait()
    own_w = pltpu.make_async_copy(w_ref, wg_ref.at[pl.ds(my * kw, kw), :],
                                  copy_sem)
    own_w.start()
    own_w.wait()
    cx = pltpu.make_async_copy(x_ref, comm_x.at[0], copy_sem)
    cx.start()
    cx.wait()
    cw = pltpu.make_async_copy(w_ref, comm_w.at[0], copy_sem)
    cw.start()
    cw.wait()

    for h in range(N_DEV - 1):
        s, r = h % 2, (h + 1) % 2
        rx = pltpu.make_async_remote_copy(
            src_ref=comm_x.at[s], dst_ref=comm_x.at[r],
            send_sem=send_x.at[s], recv_sem=recv_x.at[r],
            device_id=(right,), device_id_type=pl.DeviceIdType.MESH)
        rw = pltpu.make_async_remote_copy(
            src_ref=comm_w.at[s], dst_ref=comm_w.at[r],
            send_sem=send_w.at[s], recv_sem=recv_w.at[r],
            device_id=(right,), device_id_type=pl.DeviceIdType.MESH)
        rx.start()
        rw.start()
        rx.wait()
        rw.wait()

        origin = lax.rem(my + N_DEV - 1 - h, N_DEV)
        ox = pltpu.make_async_copy(
            comm_x.at[r], xg_ref.at[:, pl.ds(origin * kx, kx)], copy_sem)
        ox.start()
        ox.wait()
        ow = pltpu.make_async_copy(
            comm_w.at[r], wg_ref.at[pl.ds(origin * kw, kw), :], copy_sem)
        ow.start()
        ow.wait()


def _all_gather(xc, wc):
    m, kx = xc.shape
    kw, n = wc.shape
    return pl.pallas_call(
        _ag_body,
        out_shape=[
            jax.ShapeDtypeStruct((m, N_DEV * kx), xc.dtype),
            jax.ShapeDtypeStruct((N_DEV * kw, n), wc.dtype),
        ],
        in_specs=[pl.BlockSpec(memory_space=pltpu.VMEM),
                  pl.BlockSpec(memory_space=pltpu.VMEM)],
        out_specs=[pl.BlockSpec(memory_space=pl.ANY),
                   pl.BlockSpec(memory_space=pl.ANY)],
        scratch_shapes=[
            pltpu.VMEM((2, m, kx), xc.dtype),
            pltpu.VMEM((2, kw, n), wc.dtype),
            pltpu.SemaphoreType.DMA((2,)),
            pltpu.SemaphoreType.DMA((2,)),
            pltpu.SemaphoreType.DMA((2,)),
            pltpu.SemaphoreType.DMA((2,)),
            pltpu.SemaphoreType.DMA,
        ],
        compiler_params=pltpu.CompilerParams(collective_id=0),
    )(xc, wc)


def _gemm_body(s_ref, x_ref, w_ref, o_ref):
    o_ref[...] = (
        jnp.dot(x_ref[...], w_ref[...], preferred_element_type=jnp.float32)
        * s_ref[0, 0]
    )


def _gemm(s, xg, wg):
    m, k = xg.shape
    _, n = wg.shape
    return pl.pallas_call(
        _gemm_body,
        grid=(m // BM, n // BN),
        in_specs=[
            pl.BlockSpec((1, 1), lambda i, j: (0, 0),
                         memory_space=pltpu.SMEM),
            pl.BlockSpec((BM, k), lambda i, j: (i, 0)),
            pl.BlockSpec((k, BN), lambda i, j: (0, j)),
        ],
        out_specs=pl.BlockSpec((BM, BN), lambda i, j: (i, j)),
        out_shape=jax.ShapeDtypeStruct((m, n), jnp.float32),
        compiler_params=pltpu.CompilerParams(
            dimension_semantics=("parallel", "parallel")),
    )(s, xg, wg)


def kernel(x, w_mat, scale_x, scale_w):
    xc = x.astype(jnp.float8_e4m3fn)
    wc = w_mat.astype(jnp.float8_e5m2)
    xg, wg = _all_gather(xc, wc)
    if COMPUTE_IN == "bf16":
        xg = xg.astype(jnp.bfloat16)
        wg = wg.astype(jnp.bfloat16)
    s = (scale_x * scale_w).reshape(1, 1)
    return _gemm(s, xg, wg)


# device time: 419722 ns/iter; 1.4716x vs baseline; 1.4716x over previous
import jax
import jax.numpy as jnp
from jax import lax
from jax.experimental import pallas as pl
from jax.experimental.pallas import tpu as pltpu

N_DEV = 4

BM = 512
BN = 2048


def _ag_body(x_ref, w_ref, xg_ref, wg_ref,
             cxr, cxl, cwr, cwl,
             sxr, rxr, sxl, rxl, swr, rwr, swl, rwl, copy_sem):
    my = lax.axis_index("i")
    left = lax.rem(my + N_DEV - 1, N_DEV)
    right = lax.rem(my + 1, N_DEV)
    kx = x_ref.shape[1]
    kw = w_ref.shape[0]
    hx = kx // 2
    hw = kw // 2

    barrier = pltpu.get_barrier_semaphore()
    for nbr in (left, right):
        pl.semaphore_signal(barrier, inc=1, device_id=(nbr,),
                            device_id_type=pl.DeviceIdType.MESH)
    pl.semaphore_wait(barrier, 2)

    def _copy(src, dst):
        c = pltpu.make_async_copy(src, dst, copy_sem)
        c.start()
        c.wait()

    _copy(x_ref, xg_ref.at[:, pl.ds(my * kx, kx)])
    _copy(w_ref, wg_ref.at[pl.ds(my * kw, kw), :])
    _copy(x_ref.at[:, pl.ds(0, hx)], cxr.at[0])
    _copy(x_ref.at[:, pl.ds(hx, hx)], cxl.at[0])
    _copy(w_ref.at[pl.ds(0, hw), :], cwr.at[0])
    _copy(w_ref.at[pl.ds(hw, hw), :], cwl.at[0])

    for h in range(N_DEV - 1):
        s, r = h % 2, (h + 1) % 2
        rdmas = [
            pltpu.make_async_remote_copy(
                src_ref=cxr.at[s], dst_ref=cxr.at[r],
                send_sem=sxr.at[s], recv_sem=rxr.at[r],
                device_id=(right,), device_id_type=pl.DeviceIdType.MESH),
            pltpu.make_async_remote_copy(
                src_ref=cwr.at[s], dst_ref=cwr.at[r],
                send_sem=swr.at[s], recv_sem=rwr.at[r],
                device_id=(right,), device_id_type=pl.DeviceIdType.MESH),
            pltpu.make_async_remote_copy(
                src_ref=cxl.at[s], dst_ref=cxl.at[r],
                send_sem=sxl.at[s], recv_sem=rxl.at[r],
                device_id=(left,), device_id_type=pl.DeviceIdType.MESH),
            pltpu.make_async_remote_copy(
                src_ref=cwl.at[s], dst_ref=cwl.at[r],
                send_sem=swl.at[s], recv_sem=rwl.at[r],
                device_id=(left,), device_id_type=pl.DeviceIdType.MESH),
        ]
        for rd in rdmas:
            rd.start()
        for rd in rdmas:
            rd.wait()

        orig_r = lax.rem(my + N_DEV - 1 - h, N_DEV)
        orig_l = lax.rem(my + 1 + h, N_DEV)
        _copy(cxr.at[r], xg_ref.at[:, pl.ds(orig_r * kx, hx)])
        _copy(cwr.at[r], wg_ref.at[pl.ds(orig_r * kw, hw), :])
        _copy(cxl.at[r], xg_ref.at[:, pl.ds(orig_l * kx + hx, hx)])
        _copy(cwl.at[r], wg_ref.at[pl.ds(orig_l * kw + hw, hw), :])


def _all_gather(xc, wc):
    m, kx = xc.shape
    kw, n = wc.shape
    dma2 = pltpu.SemaphoreType.DMA((2,))
    return pl.pallas_call(
        _ag_body,
        out_shape=[
            jax.ShapeDtypeStruct((m, N_DEV * kx), xc.dtype),
            jax.ShapeDtypeStruct((N_DEV * kw, n), wc.dtype),
        ],
        in_specs=[pl.BlockSpec(memory_space=pltpu.VMEM),
                  pl.BlockSpec(memory_space=pltpu.VMEM)],
        out_specs=[pl.BlockSpec(memory_space=pl.ANY),
                   pl.BlockSpec(memory_space=pl.ANY)],
        scratch_shapes=[
            pltpu.VMEM((2, m, kx // 2), xc.dtype),
            pltpu.VMEM((2, m, kx // 2), xc.dtype),
            pltpu.VMEM((2, kw // 2, n), wc.dtype),
            pltpu.VMEM((2, kw // 2, n), wc.dtype),
            dma2, dma2, dma2, dma2,
            dma2, dma2, dma2, dma2,
            pltpu.SemaphoreType.DMA,
        ],
        compiler_params=pltpu.CompilerParams(collective_id=0),
    )(xc, wc)


def _gemm_body(s_ref, x_ref, w_ref, o_ref):
    o_ref[...] = (
        jnp.dot(x_ref[...], w_ref[...], preferred_element_type=jnp.float32)
        * s_ref[0, 0]
    )


def _gemm(s, xg, wg):
    m, k = xg.shape
    _, n = wg.shape
    return pl.pallas_call(
        _gemm_body,
        grid=(m // BM, n // BN),
        in_specs=[
            pl.BlockSpec((1, 1), lambda i, j: (0, 0),
                         memory_space=pltpu.SMEM),
            pl.BlockSpec((BM, k), lambda i, j: (i, 0)),
            pl.BlockSpec((k, BN), lambda i, j: (0, j)),
        ],
        out_specs=pl.BlockSpec((BM, BN), lambda i, j: (i, j)),
        out_shape=jax.ShapeDtypeStruct((m, n), jnp.float32),
        compiler_params=pltpu.CompilerParams(
            dimension_semantics=("parallel", "parallel")),
    )(s, xg, wg)


def kernel(x, w_mat, scale_x, scale_w):
    xc = x.astype(jnp.float8_e4m3fn)
    wc = w_mat.astype(jnp.float8_e5m2)
    xg, wg = _all_gather(xc, wc)
    s = (scale_x * scale_w).reshape(1, 1)
    return _gemm(s, xg, wg)
